# baseline (speedup 1.0000x reference)
import functools

import jax
import jax.numpy as jnp
from jax import lax
from jax.experimental import pallas as pl
from jax.experimental.pallas import tpu as pltpu
from jax.experimental.pallas import tpu_sc as plsc

NUM_EXPERTS = 8
TOP_K = 2
BLK = 256


def _router_body(x_ref, wr_ref, a1_ref, a2_ref, g1_ref, g2_ref):
    x = x_ref[...]
    wr = wr_ref[...]
    logits = jnp.dot(x, wr, preferred_element_type=jnp.float32)
    m = jnp.max(logits, axis=-1, keepdims=True)
    p = jnp.exp(logits - m)
    p = p / jnp.sum(p, axis=-1, keepdims=True)
    e = p.shape[-1]
    idx = lax.broadcasted_iota(jnp.int32, p.shape, 1)
    m1 = jnp.max(p, axis=-1, keepdims=True)
    a1 = jnp.min(jnp.where(p == m1, idx, e), axis=-1, keepdims=True)
    p2 = jnp.where(idx == a1, -1.0, p)
    m2 = jnp.max(p2, axis=-1, keepdims=True)
    a2 = jnp.min(jnp.where(p2 == m2, idx, e), axis=-1, keepdims=True)
    s = m1 + m2
    a1_ref[...] = a1
    a2_ref[...] = a2
    g1_ref[...] = m1 / s
    g2_ref[...] = m2 / s


def _run_router(x, w_router):
    t = x.shape[0]
    return pl.pallas_call(
        _router_body,
        out_shape=(
            jax.ShapeDtypeStruct((t, 1), jnp.int32),
            jax.ShapeDtypeStruct((t, 1), jnp.int32),
            jax.ShapeDtypeStruct((t, 1), jnp.float32),
            jax.ShapeDtypeStruct((t, 1), jnp.float32),
        ),
    )(x, w_router)


def _ffn_body(te_ref, tv_ref, xs_ref, w1_ref, b1_ref, w2_ref, b2_ref,
              gate_ref, ys_ref):
    g = pl.program_id(0)

    @pl.when(tv_ref[g] == 1)
    def _():
        x = xs_ref[...]
        h = jnp.dot(x, w1_ref[0], preferred_element_type=jnp.float32)
        h = jax.nn.gelu(h + b1_ref[0])
        y = jnp.dot(h, w2_ref[0], preferred_element_type=jnp.float32)
        ys_ref[...] = (y + b2_ref[0]) * gate_ref[...]


def _run_ffn(tile_expert, tile_valid, xs, w1, b1, w2, b2, gate_col):
    pad, h = xs.shape
    e, _, f = w1.shape
    g_max = pad // BLK
    grid_spec = pltpu.PrefetchScalarGridSpec(
        num_scalar_prefetch=2,
        grid=(g_max,),
        in_specs=[
            pl.BlockSpec((BLK, h), lambda g, te, tv: (g, 0)),
            pl.BlockSpec((1, h, f), lambda g, te, tv: (te[g], 0, 0)),
            pl.BlockSpec((1, 1, f), lambda g, te, tv: (te[g], 0, 0)),
            pl.BlockSpec((1, f, h), lambda g, te, tv: (te[g], 0, 0)),
            pl.BlockSpec((1, 1, h), lambda g, te, tv: (te[g], 0, 0)),
            pl.BlockSpec((BLK, 1), lambda g, te, tv: (g, 0)),
        ],
        out_specs=pl.BlockSpec((BLK, h), lambda g, te, tv: (g, 0)),
    )
    return pl.pallas_call(
        _ffn_body,
        grid_spec=grid_spec,
        out_shape=jax.ShapeDtypeStruct((pad, h), jnp.float32),
    )(tile_expert, tile_valid, xs, w1, b1.reshape(b1.shape[0], 1, f),
      w2, b2.reshape(b2.shape[0], 1, h), gate_col)


@functools.cache
def _make_sc_gather(n_rows, h):
    info = plsc.get_sparse_core_info()
    nw = info.num_cores * info.num_subcores
    per_w = n_rows // nw
    ch = 64
    n_ch = per_w // ch
    mesh = plsc.VectorSubcoreMesh(core_axis_name="c", subcore_axis_name="s")

    @functools.partial(
        pl.kernel, mesh=mesh,
        out_type=jax.ShapeDtypeStruct((n_rows, h), jnp.float32),
        scratch_types=[
            pltpu.VMEM((per_w,), jnp.int32),
            pltpu.VMEM((ch, h), jnp.float32),
            pltpu.SemaphoreType.DMA,
        ],
    )
    def k(tab_hbm, idx_hbm, out_hbm, idx_v, rows_v, sem):
        wid = lax.axis_index("s") * info.num_cores + lax.axis_index("c")
        base = wid * per_w
        pltpu.sync_copy(idx_hbm.at[pl.ds(base, per_w)], idx_v)
        for c in range(n_ch):
            pltpu.async_copy(
                tab_hbm.at[idx_v.at[pl.ds(c * ch, ch)]], rows_v, sem).wait()
            pltpu.sync_copy(rows_v, out_hbm.at[pl.ds(base + c * ch, ch)])

    return k


@functools.cache
def _make_sc_combine(n_tok, h):
    info = plsc.get_sparse_core_info()
    nw = info.num_cores * info.num_subcores
    per_w = n_tok // nw
    ch = 32
    n_ch = per_w // ch
    mesh = plsc.VectorSubcoreMesh(core_axis_name="c", subcore_axis_name="s")

    @functools.partial(
        pl.kernel, mesh=mesh,
        out_type=jax.ShapeDtypeStruct((n_tok, h), jnp.float32),
        scratch_types=[
            pltpu.VMEM((per_w,), jnp.int32),
            pltpu.VMEM((per_w,), jnp.int32),
            pltpu.VMEM((ch, h), jnp.float32),
            pltpu.VMEM((ch, h), jnp.float32),
            pltpu.SemaphoreType.DMA,
            pltpu.SemaphoreType.DMA,
        ],
    )
    def k(ys_hbm, p0_hbm, p1_hbm, out_hbm, i0_v, i1_v, r0_v, r1_v, s0, s1):
        wid = lax.axis_index("s") * info.num_cores + lax.axis_index("c")
        base = wid * per_w
        pltpu.sync_copy(p0_hbm.at[pl.ds(base, per_w)], i0_v)
        pltpu.sync_copy(p1_hbm.at[pl.ds(base, per_w)], i1_v)
        for c in range(n_ch):
            cp0 = pltpu.async_copy(
                ys_hbm.at[i0_v.at[pl.ds(c * ch, ch)]], r0_v, s0)
            cp1 = pltpu.async_copy(
                ys_hbm.at[i1_v.at[pl.ds(c * ch, ch)]], r1_v, s1)
            cp0.wait()
            cp1.wait()
            for i in range(ch):
                def vbody(j, _, i=i):
                    sl = pl.ds(j * 16, 16)
                    r0_v[i, sl] = r0_v[i, sl] + r1_v[i, sl]
                    return 0
                lax.fori_loop(0, h // 16, vbody, 0)
            pltpu.sync_copy(r0_v, out_hbm.at[pl.ds(base + c * ch, ch)])

    return k


def _dispatch_tables(e1, e2, g1, g2, pad):
    t = e1.shape[0]
    flat_e = jnp.stack([e1, e2], axis=1).reshape(-1)
    oh = (flat_e[:, None] == jnp.arange(NUM_EXPERTS, dtype=jnp.int32)
          ).astype(jnp.int32)
    cum = jnp.cumsum(oh, axis=0)
    rank_within = jnp.take_along_axis(cum, flat_e[:, None], axis=1)[:, 0] - 1
    counts = cum[-1]
    tiles_e = (counts + BLK - 1) // BLK
    aligned_off = jnp.concatenate(
        [jnp.zeros((1,), jnp.int32), jnp.cumsum(tiles_e)]) * BLK
    pos = aligned_off[flat_e] + rank_within
    pair_tok = jnp.arange(TOP_K * t, dtype=jnp.int32) // TOP_K
    sorted_tok = jnp.zeros((pad,), jnp.int32).at[pos].set(pair_tok)
    gate_flat = jnp.stack([g1, g2], axis=1).reshape(-1)
    gate_sorted = jnp.zeros((pad,), jnp.float32).at[pos].set(gate_flat)
    total = aligned_off[NUM_EXPERTS]
    g_max = pad // BLK
    gstart = jnp.arange(g_max, dtype=jnp.int32) * BLK
    tile_expert = jnp.searchsorted(
        aligned_off[1:], jnp.minimum(gstart, total - 1),
        side="right").astype(jnp.int32)
    tile_valid = (gstart < total).astype(jnp.int32)
    return pos, sorted_tok, gate_sorted, tile_expert, tile_valid


def kernel(hidden_states, W_router, W1, b1, W2, b2):
    b, s, h = hidden_states.shape
    t = b * s
    pad = (TOP_K * t // BLK + NUM_EXPERTS) * BLK
    x = hidden_states.reshape(t, h)

    a1, a2, g1, g2 = _run_router(x, W_router)
    pos, sorted_tok, gate_sorted, tile_expert, tile_valid = _dispatch_tables(
        a1[:, 0], a2[:, 0], g1[:, 0], g2[:, 0], pad)

    xs = _make_sc_gather(pad, h)(x, sorted_tok)
    ysw = _run_ffn(tile_expert, tile_valid, xs, W1, b1, W2, b2,
                   gate_sorted.reshape(pad, 1))
    pos2 = pos.reshape(t, TOP_K)
    out = _make_sc_combine(t, h)(ysw, pos2[:, 0], pos2[:, 1])
    return out.reshape(b, s, h)


# speedup vs baseline: 1.0150x; 1.0150x over previous
import functools

import jax
import jax.numpy as jnp
from jax import lax
from jax.experimental import pallas as pl
from jax.experimental.pallas import tpu as pltpu
from jax.experimental.pallas import tpu_sc as plsc

NUM_EXPERTS = 8
TOP_K = 2
BLK = 256


def _router_body(x_ref, wr_ref, a1_ref, a2_ref, g1_ref, g2_ref):
    x = x_ref[...]
    wr = wr_ref[...]
    logits = jnp.dot(x, wr, preferred_element_type=jnp.float32)
    m = jnp.max(logits, axis=-1, keepdims=True)
    p = jnp.exp(logits - m)
    p = p / jnp.sum(p, axis=-1, keepdims=True)
    e = p.shape[-1]
    idx = lax.broadcasted_iota(jnp.int32, p.shape, 1)
    m1 = jnp.max(p, axis=-1, keepdims=True)
    a1 = jnp.min(jnp.where(p == m1, idx, e), axis=-1, keepdims=True)
    p2 = jnp.where(idx == a1, -1.0, p)
    m2 = jnp.max(p2, axis=-1, keepdims=True)
    a2 = jnp.min(jnp.where(p2 == m2, idx, e), axis=-1, keepdims=True)
    s = m1 + m2
    a1_ref[...] = a1
    a2_ref[...] = a2
    g1_ref[...] = m1 / s
    g2_ref[...] = m2 / s


def _run_router(x, w_router):
    t = x.shape[0]
    return pl.pallas_call(
        _router_body,
        out_shape=(
            jax.ShapeDtypeStruct((t, 1), jnp.int32),
            jax.ShapeDtypeStruct((t, 1), jnp.int32),
            jax.ShapeDtypeStruct((t, 1), jnp.float32),
            jax.ShapeDtypeStruct((t, 1), jnp.float32),
        ),
    )(x, w_router)


def _ffn_body(te_ref, tv_ref, xs_ref, w1_ref, b1_ref, w2_ref, b2_ref,
              gate_ref, ys_ref):
    g = pl.program_id(0)

    @pl.when(tv_ref[g] == 1)
    def _():
        x = xs_ref[...]
        h = jnp.dot(x, w1_ref[0], preferred_element_type=jnp.float32)
        h = jax.nn.gelu(h + b1_ref[0])
        y = jnp.dot(h, w2_ref[0], preferred_element_type=jnp.float32)
        ys_ref[...] = (y + b2_ref[0]) * gate_ref[...]


def _run_ffn(tile_expert, tile_valid, xs, w1, b1, w2, b2, gate_col):
    pad, h = xs.shape
    e, _, f = w1.shape
    g_max = pad // BLK
    grid_spec = pltpu.PrefetchScalarGridSpec(
        num_scalar_prefetch=2,
        grid=(g_max,),
        in_specs=[
            pl.BlockSpec((BLK, h), lambda g, te, tv: (g, 0)),
            pl.BlockSpec((1, h, f), lambda g, te, tv: (te[g], 0, 0)),
            pl.BlockSpec((1, 1, f), lambda g, te, tv: (te[g], 0, 0)),
            pl.BlockSpec((1, f, h), lambda g, te, tv: (te[g], 0, 0)),
            pl.BlockSpec((1, 1, h), lambda g, te, tv: (te[g], 0, 0)),
            pl.BlockSpec((BLK, 1), lambda g, te, tv: (g, 0)),
        ],
        out_specs=pl.BlockSpec((BLK, h), lambda g, te, tv: (g, 0)),
    )
    return pl.pallas_call(
        _ffn_body,
        grid_spec=grid_spec,
        out_shape=jax.ShapeDtypeStruct((pad, h), jnp.float32),
    )(tile_expert, tile_valid, xs, w1, b1.reshape(b1.shape[0], 1, f),
      w2, b2.reshape(b2.shape[0], 1, h), gate_col)


@functools.cache
def _make_sc_gather(n_rows, h):
    info = plsc.get_sparse_core_info()
    nw = info.num_cores * info.num_subcores
    per_w = n_rows // nw
    ch = 48
    n_ch = per_w // ch
    mesh = plsc.VectorSubcoreMesh(core_axis_name="c", subcore_axis_name="s")

    @functools.partial(
        pl.kernel, mesh=mesh,
        out_type=jax.ShapeDtypeStruct((n_rows, h), jnp.float32),
        scratch_types=[
            pltpu.VMEM((per_w,), jnp.int32),
            pltpu.VMEM((ch, h), jnp.float32),
            pltpu.VMEM((ch, h), jnp.float32),
            pltpu.SemaphoreType.DMA,
            pltpu.SemaphoreType.DMA,
            pltpu.SemaphoreType.DMA,
            pltpu.SemaphoreType.DMA,
        ],
    )
    def k(tab_hbm, idx_hbm, out_hbm, idx_v, r0, r1, g0, g1, w0, w1):
        wid = lax.axis_index("s") * info.num_cores + lax.axis_index("c")
        base = wid * per_w
        pltpu.sync_copy(idx_hbm.at[pl.ds(base, per_w)], idx_v)
        bufs, gsem, wsem = (r0, r1), (g0, g1), (w0, w1)

        def g_issue(c):
            b = c & 1
            return pltpu.async_copy(
                tab_hbm.at[idx_v.at[pl.ds(c * ch, ch)]], bufs[b], gsem[b])

        gcp = {0: g_issue(0)}
        wcp = {}
        for c in range(n_ch):
            b = c & 1
            gcp[c].wait()
            if c + 1 < n_ch:
                if c >= 1:
                    wcp[c - 1].wait()
                gcp[c + 1] = g_issue(c + 1)
            wcp[c] = pltpu.async_copy(
                bufs[b], out_hbm.at[pl.ds(base + c * ch, ch)], wsem[b])
        if n_ch >= 2:
            wcp[n_ch - 2].wait()
        wcp[n_ch - 1].wait()

    return k


@functools.cache
def _make_sc_combine(n_tok, h):
    info = plsc.get_sparse_core_info()
    nw = info.num_cores * info.num_subcores
    per_w = n_tok // nw
    ch = 32
    n_ch = per_w // ch
    assert n_ch == 2 and per_w % ch == 0
    mesh = plsc.VectorSubcoreMesh(core_axis_name="c", subcore_axis_name="s")

    @functools.partial(
        pl.kernel, mesh=mesh,
        out_type=jax.ShapeDtypeStruct((n_tok, h), jnp.float32),
        scratch_types=[
            pltpu.VMEM((per_w,), jnp.int32),
            pltpu.VMEM((per_w,), jnp.int32),
            pltpu.VMEM((ch, h), jnp.float32),
            pltpu.VMEM((ch, h), jnp.float32),
            pltpu.VMEM((ch, h), jnp.float32),
            pltpu.SemaphoreType.DMA,
            pltpu.SemaphoreType.DMA,
            pltpu.SemaphoreType.DMA,
            pltpu.SemaphoreType.DMA,
        ],
    )
    def k(ys_hbm, p0_hbm, p1_hbm, out_hbm,
          i0_v, i1_v, r0a, r0b, r1_v, sa, sb, s1, sw):
        wid = lax.axis_index("s") * info.num_cores + lax.axis_index("c")
        base = wid * per_w
        pltpu.sync_copy(p0_hbm.at[pl.ds(base, per_w)], i0_v)
        pltpu.sync_copy(p1_hbm.at[pl.ds(base, per_w)], i1_v)
        bufs, gsem = (r0a, r0b), (sa, sb)

        def add_rows(dst, src):
            for i in range(ch):
                def vbody(j, _, i=i):
                    for u in range(4):
                        sl = pl.ds((j * 4 + u) * 16, 16)
                        dst[i, sl] = dst[i, sl] + src[i, sl]
                    return 0
                lax.fori_loop(0, h // 64, vbody, 0)

        cpa = pltpu.async_copy(ys_hbm.at[i0_v.at[pl.ds(0, ch)]], r0a, sa)
        cp1 = pltpu.async_copy(ys_hbm.at[i1_v.at[pl.ds(0, ch)]], r1_v, s1)
        cpb = pltpu.async_copy(ys_hbm.at[i0_v.at[pl.ds(ch, ch)]], r0b, sb)
        cpa.wait()
        cp1.wait()
        add_rows(r0a, r1_v)
        cp1b = pltpu.async_copy(ys_hbm.at[i1_v.at[pl.ds(ch, ch)]], r1_v, s1)
        w0 = pltpu.async_copy(r0a, out_hbm.at[pl.ds(base, ch)], sw)
        cpb.wait()
        cp1b.wait()
        add_rows(r0b, r1_v)
        w0.wait()
        w1 = pltpu.async_copy(r0b, out_hbm.at[pl.ds(base + ch, ch)], sw)
        w1.wait()

    return k


def _dispatch_tables(e1, e2, g1, g2, pad):
    t = e1.shape[0]
    flat_e = jnp.stack([e1, e2], axis=1).reshape(-1)
    oh = (flat_e[:, None] == jnp.arange(NUM_EXPERTS, dtype=jnp.int32)
          ).astype(jnp.int32)
    cum = jnp.cumsum(oh, axis=0)
    rank_within = jnp.take_along_axis(cum, flat_e[:, None], axis=1)[:, 0] - 1
    counts = cum[-1]
    tiles_e = (counts + BLK - 1) // BLK
    aligned_off = jnp.concatenate(
        [jnp.zeros((1,), jnp.int32), jnp.cumsum(tiles_e)]) * BLK
    pos = aligned_off[flat_e] + rank_within
    pair_tok = jnp.arange(TOP_K * t, dtype=jnp.int32) // TOP_K
    sorted_tok = jnp.zeros((pad,), jnp.int32).at[pos].set(pair_tok)
    gate_flat = jnp.stack([g1, g2], axis=1).reshape(-1)
    gate_sorted = jnp.zeros((pad,), jnp.float32).at[pos].set(gate_flat)
    total = aligned_off[NUM_EXPERTS]
    g_max = pad // BLK
    gstart = jnp.arange(g_max, dtype=jnp.int32) * BLK
    tile_expert = jnp.searchsorted(
        aligned_off[1:], jnp.minimum(gstart, total - 1),
        side="right").astype(jnp.int32)
    tile_valid = (gstart < total).astype(jnp.int32)
    return pos, sorted_tok, gate_sorted, tile_expert, tile_valid


def kernel(hidden_states, W_router, W1, b1, W2, b2):
    b, s, h = hidden_states.shape
    t = b * s
    pad = (TOP_K * t // BLK + NUM_EXPERTS) * BLK
    x = hidden_states.reshape(t, h)

    a1, a2, g1, g2 = _run_router(x, W_router)
    pos, sorted_tok, gate_sorted, tile_expert, tile_valid = _dispatch_tables(
        a1[:, 0], a2[:, 0], g1[:, 0], g2[:, 0], pad)

    xs = _make_sc_gather(pad, h)(x, sorted_tok)
    ysw = _run_ffn(tile_expert, tile_valid, xs, W1, b1, W2, b2,
                   gate_sorted.reshape(pad, 1))
    pos2 = pos.reshape(t, TOP_K)
    out = _make_sc_combine(t, h)(ysw, pos2[:, 0], pos2[:, 1])
    return out.reshape(b, s, h)
